# baseline (device time: 281578 ns/iter reference)
import jax
import jax.numpy as jnp
from jax import lax
from jax.experimental import pallas as pl
from jax.experimental.pallas import tpu as pltpu

P = 32
B = 2
SQ = 512
SKV = 512
H_LOC = 8
DH = 64
D = 768
HD = H_LOC * DH
ROWS = B * SQ
CHUNK = ROWS // P

_DIT = getattr(pl, "DeviceIdType", None) or pltpu.DeviceIdType
_signal = getattr(pl, "semaphore_signal", None) or pltpu.semaphore_signal
_swait = getattr(pl, "semaphore_wait", None) or pltpu.semaphore_wait
_CompilerParams = getattr(pltpu, "CompilerParams", None) or pltpu.TPUCompilerParams


def _body(x_ref, wq_ref, k_ref, v_ref, wo_ref, out_ref,
          recv_buf, send_sems, recv_sems, credit_sem):
    my = lax.axis_index("i")
    left = lax.rem(my + P - 1, P)
    right = lax.rem(my + 1, P)

    barrier = pltpu.get_barrier_semaphore()
    _signal(barrier, inc=1, device_id=(left,), device_id_type=_DIT.MESH)
    _signal(barrier, inc=1, device_id=(right,), device_id_type=_DIT.MESH)
    _swait(barrier, 2)

    ri = lax.broadcasted_iota(jnp.int32, (SQ, SKV), 0)
    ci = lax.broadcasted_iota(jnp.int32, (SQ, SKV), 1)
    mask = ((ri // 64) % 4) == ((ci // 64) % 4)
    for b in range(B):
        xb = x_ref[b * SQ:(b + 1) * SQ, :]
        q = jnp.dot(xb, wq_ref[:, :], preferred_element_type=jnp.float32)
        acc = jnp.zeros((SQ, D), jnp.float32)
        for h in range(H_LOC):
            g = b * H_LOC + h
            s = jnp.dot(q[:, h * DH:(h + 1) * DH], k_ref[g],
                        preferred_element_type=jnp.float32)
            s = jnp.where(mask, s * 0.125, -1e9)
            m = jnp.max(s, axis=-1, keepdims=True)
            w = jnp.exp(s - m)
            w = w / jnp.sum(w, axis=-1, keepdims=True)
            ctx = jnp.dot(w, v_ref[g], preferred_element_type=jnp.float32)
            acc = acc + jnp.dot(ctx, wo_ref[h * DH:(h + 1) * DH, :],
                                preferred_element_type=jnp.float32)
        out_ref[b * SQ:(b + 1) * SQ, :] = acc


    def rs_step(s, carry):
        slot = lax.rem(s, 2)
        send_c = lax.rem(my - s + P, P)
        recv_c = lax.rem(my - s - 1 + 2 * P, P)

        @pl.when(s >= 2)
        def _():
            _swait(credit_sem, 1)

        rdma = pltpu.make_async_remote_copy(
            src_ref=out_ref.at[pl.ds(send_c * CHUNK, CHUNK), :],
            dst_ref=recv_buf.at[slot],
            send_sem=send_sems.at[slot],
            recv_sem=recv_sems.at[slot],
            device_id=(right,),
            device_id_type=_DIT.MESH,
        )
        rdma.start()
        rdma.wait()
        out_ref[pl.ds(recv_c * CHUNK, CHUNK), :] = (
            out_ref[pl.ds(recv_c * CHUNK, CHUNK), :] + recv_buf[slot]
        )
        _signal(credit_sem, inc=1, device_id=(left,), device_id_type=_DIT.MESH)
        return carry

    lax.fori_loop(0, P - 1, rs_step, 0)
    _swait(credit_sem, 2)

    def ag_step(s, carry):
        slot = lax.rem(s, 2)
        send_c = lax.rem(my + 1 - s + 2 * P, P)

        @pl.when(s >= 2)
        def _():
            _swait(credit_sem, 1)

        rdma = pltpu.make_async_remote_copy(
            src_ref=out_ref.at[pl.ds(send_c * CHUNK, CHUNK), :],
            dst_ref=out_ref.at[pl.ds(send_c * CHUNK, CHUNK), :],
            send_sem=send_sems.at[slot],
            recv_sem=recv_sems.at[slot],
            device_id=(right,),
            device_id_type=_DIT.MESH,
        )
        rdma.start()
        rdma.wait()
        _signal(credit_sem, inc=1, device_id=(left,), device_id_type=_DIT.MESH)
        return carry

    lax.fori_loop(0, P - 1, ag_step, 0)
    _swait(credit_sem, 2)


def kernel(x, Wq, K_ext, V_ext, Wo):
    my = lax.axis_index("i")
    wq_my = lax.dynamic_slice(Wq, (0, my * HD), (D, HD))
    wo_my = lax.dynamic_slice(Wo, (my * HD, 0), (HD, D))
    x2d = x.reshape(ROWS, D)
    k_in = K_ext.transpose(0, 2, 3, 1).reshape(B * H_LOC, DH, SKV)
    v_in = V_ext.transpose(0, 2, 1, 3).reshape(B * H_LOC, SKV, DH)

    out = pl.pallas_call(
        _body,
        out_shape=jax.ShapeDtypeStruct((ROWS, D), jnp.float32),
        in_specs=[pl.BlockSpec(memory_space=pltpu.VMEM)] * 5,
        out_specs=pl.BlockSpec(memory_space=pltpu.VMEM),
        scratch_shapes=[
            pltpu.VMEM((2, CHUNK, D), jnp.float32),
            pltpu.SemaphoreType.DMA((2,)),
            pltpu.SemaphoreType.DMA((2,)),
            pltpu.SemaphoreType.REGULAR,
        ],
        compiler_params=_CompilerParams(collective_id=0),
    )(x2d, wq_my, k_in, v_in, wo_my)
    return out.reshape(B, SQ, D)


# device time: 81713 ns/iter; 3.4459x vs baseline; 3.4459x over previous
import jax
import jax.numpy as jnp
from jax import lax
from jax.experimental import pallas as pl
from jax.experimental.pallas import tpu as pltpu

P = 32
LOGP = 5
B = 2
SQ = 512
SKV = 512
H_LOC = 8
DH = 64
D = 768
HD = H_LOC * DH
ROWS = B * SQ

_RS_LEN = [512 >> k for k in range(LOGP)]
_RS_OFF = [sum(_RS_LEN[:k]) for k in range(LOGP)]
_RS_TOT = sum(_RS_LEN)
_AG_LEN = [1024 >> (k + 1) for k in range(LOGP - 1, -1, -1)]
_AG_OFF = [sum(_AG_LEN[:j]) for j in range(LOGP)]

_DIT = getattr(pl, "DeviceIdType", None) or pltpu.DeviceIdType
_signal = getattr(pl, "semaphore_signal", None) or pltpu.semaphore_signal
_swait = getattr(pl, "semaphore_wait", None) or pltpu.semaphore_wait
_CompilerParams = getattr(pltpu, "CompilerParams", None) or pltpu.TPUCompilerParams


def _body(x_ref, wq_ref, k_ref, v_ref, wo_ref, out_ref,
          stage_buf, rs_buf, ag_buf, send_sems, recv_sems):
    my = lax.axis_index("i")

    barrier = pltpu.get_barrier_semaphore()
    for k in range(LOGP):
        _signal(barrier, inc=1, device_id=(my ^ (1 << k),),
                device_id_type=_DIT.MESH)
    _swait(barrier, LOGP)

    ri = lax.broadcasted_iota(jnp.int32, (SQ, SKV), 0)
    ci = lax.broadcasted_iota(jnp.int32, (SQ, SKV), 1)
    mask = ((ri // 64) % 4) == ((ci // 64) % 4)
    for b in range(B):
        xb = x_ref[b * SQ:(b + 1) * SQ, :]
        q = jnp.dot(xb, wq_ref[:, :], preferred_element_type=jnp.float32)
        acc = jnp.zeros((SQ, D), jnp.float32)
        for h in range(H_LOC):
            g = b * H_LOC + h
            s = jnp.dot(q[:, h * DH:(h + 1) * DH], k_ref[g],
                        preferred_element_type=jnp.float32)
            s = jnp.where(mask, s * 0.125, -1e9)
            m = jnp.max(s, axis=-1, keepdims=True)
            w = jnp.exp(s - m)
            w = w / jnp.sum(w, axis=-1, keepdims=True)
            ctx = jnp.dot(w, v_ref[g], preferred_element_type=jnp.float32)
            acc = acc + jnp.dot(ctx, wo_ref[h * DH:(h + 1) * DH, :],
                                preferred_element_type=jnp.float32)
        out_ref[b * SQ:(b + 1) * SQ, :] = acc

    base = my * 0
    rs_rdmas = []
    for k in range(LOGP):
        half = _RS_LEN[k]
        bit = lax.rem(lax.div(my, 1 << k), 2)
        partner = my ^ (1 << k)
        keep_base = base + bit * half
        send_base = base + (1 - bit) * half
        stage_buf[pl.ds(_RS_OFF[k], half), :] = (
            out_ref[pl.ds(send_base, half), :].astype(jnp.bfloat16)
        )
        rdma = pltpu.make_async_remote_copy(
            src_ref=stage_buf.at[pl.ds(_RS_OFF[k], half), :],
            dst_ref=rs_buf.at[pl.ds(_RS_OFF[k], half), :],
            send_sem=send_sems.at[k],
            recv_sem=recv_sems.at[k],
            device_id=(partner,),
            device_id_type=_DIT.MESH,
        )
        rdma.start()
        rs_rdmas.append(rdma)
        rdma.wait_recv()
        out_ref[pl.ds(keep_base, half), :] = (
            out_ref[pl.ds(keep_base, half), :]
            + rs_buf[pl.ds(_RS_OFF[k], half), :].astype(jnp.float32)
        )
        base = keep_base

    for r in rs_rdmas:
        r.wait_send()

    ag_rdmas = []
    for j, k in enumerate(range(LOGP - 1, -1, -1)):
        seg = _AG_LEN[j]
        bit = lax.rem(lax.div(my, 1 << k), 2)
        partner = my ^ (1 << k)
        sem = LOGP + j
        stage_buf[pl.ds(_AG_OFF[j], seg), :] = (
            out_ref[pl.ds(base, seg), :].astype(jnp.bfloat16)
        )
        rdma = pltpu.make_async_remote_copy(
            src_ref=stage_buf.at[pl.ds(_AG_OFF[j], seg), :],
            dst_ref=ag_buf.at[pl.ds(_AG_OFF[j], seg), :],
            send_sem=send_sems.at[sem],
            recv_sem=recv_sems.at[sem],
            device_id=(partner,),
            device_id_type=_DIT.MESH,
        )
        rdma.start()
        ag_rdmas.append(rdma)
        rdma.wait_recv()
        other_base = base + (1 - 2 * bit) * seg
        out_ref[pl.ds(other_base, seg), :] = (
            ag_buf[pl.ds(_AG_OFF[j], seg), :].astype(jnp.float32)
        )
        base = base - bit * seg

    for r in ag_rdmas:
        r.wait_send()


def kernel(x, Wq, K_ext, V_ext, Wo):
    my = lax.axis_index("i")
    wq_my = lax.dynamic_slice(Wq, (0, my * HD), (D, HD))
    wo_my = lax.dynamic_slice(Wo, (my * HD, 0), (HD, D))
    x2d = x.reshape(ROWS, D)
    k_in = K_ext.transpose(0, 2, 3, 1).reshape(B * H_LOC, DH, SKV)
    v_in = V_ext.transpose(0, 2, 1, 3).reshape(B * H_LOC, SKV, DH)

    out = pl.pallas_call(
        _body,
        out_shape=jax.ShapeDtypeStruct((ROWS, D), jnp.float32),
        in_specs=[pl.BlockSpec(memory_space=pltpu.VMEM)] * 5,
        out_specs=pl.BlockSpec(memory_space=pltpu.VMEM),
        scratch_shapes=[
            pltpu.VMEM((_RS_TOT, D), jnp.bfloat16),
            pltpu.VMEM((_RS_TOT, D), jnp.bfloat16),
            pltpu.VMEM((_RS_TOT, D), jnp.bfloat16),
            pltpu.SemaphoreType.DMA((2 * LOGP,)),
            pltpu.SemaphoreType.DMA((2 * LOGP,)),
        ],
        compiler_params=_CompilerParams(collective_id=0),
    )(x2d, wq_my, k_in, v_in, wo_my)
    return out.reshape(B, SQ, D)
